# Initial kernel scaffold; baseline (speedup 1.0000x reference)
import functools

import jax
import jax.numpy as jnp
from jax import lax
from jax.experimental import pallas as pl
from jax.experimental.pallas import tpu as pltpu
from jax.experimental.pallas import tpu_sc as plsc

NC = 2
NS = 16
L = 16
NW = NC * NS

N_NODES = 10000
N_EDGES = 320000
D_EDGE = 16
D_NODE = 128
HID = 64

NPAD = 10240
STRIPE = NPAD // NS
CHUNK = 128
N_CHUNKS = N_EDGES // CHUNK
CHUNKS_PER_TILE = -(-N_CHUNKS // NW)
N_CHUNKS_PAD = CHUNKS_PER_TILE * NW

_mesh = plsc.VectorSubcoreMesh(core_axis_name="c", subcore_axis_name="s")


@functools.partial(
    pl.kernel,
    out_type=(
        jax.ShapeDtypeStruct((NC, NPAD, L), jnp.float32),
        jax.ShapeDtypeStruct((NC, NPAD, L), jnp.float32),
    ),
    mesh=_mesh,
    scratch_types=[
        pltpu.VMEM((1, CHUNK), jnp.int32),
        pltpu.VMEM((CHUNK, L), jnp.float32),
        pltpu.VMEM((NPAD,), jnp.float32),
        pltpu.VMEM((STRIPE, L), jnp.float32),
        pltpu.VMEM((NS, STRIPE), jnp.float32),
        pltpu.VMEM_SHARED((NPAD, L), jnp.float32),
        pltpu.VMEM_SHARED((NS, NPAD), jnp.float32),
    ],
)
def _sc_scatter(colp_hbm, edge_hbm, sums_out, cnt_out,
                idxbuf, databuf, cnt_v, out16, stage_v, sums_sh, stage_sh):
    c = lax.axis_index("c")
    s = lax.axis_index("s")
    w = s * NC + c
    z16 = jnp.zeros((L,), jnp.float32)
    ones16 = jnp.ones((L,), jnp.float32)

    def zero_body(n, carry):
        out16[n, :] = z16
        cnt_v[pl.ds(n * L, L)] = z16
        return carry
    lax.fori_loop(0, STRIPE, zero_body, 0)
    pltpu.sync_copy(out16, sums_sh.at[pl.ds(s * STRIPE, STRIPE)])
    plsc.subcore_barrier()

    def chunk_body(j, carry):
        chunk = j * NW + w
        data_chunk = jnp.minimum(chunk, N_CHUNKS - 1)
        pltpu.sync_copy(colp_hbm.at[chunk], idxbuf.at[0])
        pltpu.sync_copy(edge_hbm.at[pl.ds(data_chunk * CHUNK, CHUNK)], databuf)
        pltpu.sync_copy(databuf, sums_sh.at[idxbuf.at[0]], add=True)
        for k in range(CHUNK // L):
            idxv = idxbuf[0, pl.ds(k * L, L)]
            plsc.addupdate_scatter(cnt_v, [idxv], ones16)
        return carry
    lax.fori_loop(0, CHUNKS_PER_TILE, chunk_body, 0)
    plsc.subcore_barrier()

    pltpu.sync_copy(cnt_v, stage_sh.at[s])
    plsc.subcore_barrier()
    for t in range(NS):
        pltpu.sync_copy(stage_sh.at[t, pl.ds(s * STRIPE, STRIPE)],
                        stage_v.at[t])

    def red_body(k, carry):
        acc = z16
        for t in range(NS):
            acc = acc + stage_v[t, pl.ds(k * L, L)]
        cnt_v[pl.ds(k * L, L)] = acc
        return carry
    lax.fori_loop(0, STRIPE // L, red_body, 0)

    def exp_body(n, carry):
        nv = jnp.full((L,), n, jnp.int32)
        out16[n, :] = plsc.load_gather(cnt_v, [nv])
        return carry
    lax.fori_loop(0, STRIPE, exp_body, 0)

    pltpu.sync_copy(sums_sh.at[pl.ds(s * STRIPE, STRIPE)],
                    sums_out.at[c, pl.ds(s * STRIPE, STRIPE)])
    pltpu.sync_copy(out16, cnt_out.at[c, pl.ds(s * STRIPE, STRIPE)])


def _mlp_body(x_ref, sums_ref, cnt_ref, w1x_ref, w1a_ref, b1_ref, w2_ref,
              b2_ref, out_ref):
    x = x_ref[...]
    ssum = sums_ref[0] + sums_ref[1]
    cnt = cnt_ref[0] + cnt_ref[1]
    agg = ssum / jnp.maximum(cnt, 1.0)
    h = jnp.dot(x, w1x_ref[...], preferred_element_type=jnp.float32)
    h = h + jnp.dot(agg, w1a_ref[...], preferred_element_type=jnp.float32)
    h = jnp.maximum(h + b1_ref[...], 0.0)
    out_ref[...] = (x + jnp.dot(h, w2_ref[...],
                                preferred_element_type=jnp.float32)
                    + b2_ref[...])


_NB = 2000


def _tc_mlp(x, sums, cnt16, w1x, w1a, b1, w2, b2):
    return pl.pallas_call(
        _mlp_body,
        out_shape=jax.ShapeDtypeStruct((N_NODES, D_NODE), jnp.float32),
        grid=(N_NODES // _NB,),
        in_specs=[
            pl.BlockSpec((_NB, D_NODE), lambda i: (i, 0)),
            pl.BlockSpec((NC, _NB, L), lambda i: (0, i, 0)),
            pl.BlockSpec((NC, _NB, L), lambda i: (0, i, 0)),
            pl.BlockSpec((D_NODE, HID), lambda i: (0, 0)),
            pl.BlockSpec((L, HID), lambda i: (0, 0)),
            pl.BlockSpec((1, HID), lambda i: (0, 0)),
            pl.BlockSpec((HID, D_NODE), lambda i: (0, 0)),
            pl.BlockSpec((1, D_NODE), lambda i: (0, 0)),
        ],
        out_specs=pl.BlockSpec((_NB, D_NODE), lambda i: (i, 0)),
    )(x, sums, cnt16, w1x, w1a, b1, w2, b2)


@jax.jit
def kernel(x, edge_index, edge_attr, u, batch, W1, b1, W2, b2):
    col = edge_index[1]
    n_pad_idx = N_CHUNKS_PAD * CHUNK - N_EDGES
    pad_idx = N_NODES + (jnp.arange(n_pad_idx, dtype=jnp.int32)
                         % (NPAD - N_NODES))
    colp = jnp.concatenate([col, pad_idx]).reshape(N_CHUNKS_PAD, CHUNK)
    sums, cnt16 = _sc_scatter(colp, edge_attr)
    return _tc_mlp(x, sums, cnt16, W1[:D_NODE], W1[D_NODE:],
                   b1.reshape(1, HID), W2, b2.reshape(1, D_NODE))


# speedup vs baseline: 4.7363x; 4.7363x over previous
import functools

import jax
import jax.numpy as jnp
from jax import lax
from jax.experimental import pallas as pl
from jax.experimental.pallas import tpu as pltpu
from jax.experimental.pallas import tpu_sc as plsc

NC = 2
NS = 16
L = 16
NW = NC * NS

N_NODES = 10000
N_EDGES = 320000
D_EDGE = 16
D_NODE = 128
HID = 64

NPAD = 10240
STRIPE = NPAD // NS
CHUNK = 128
N_CHUNKS = N_EDGES // CHUNK
CHUNKS_PER_TILE = -(-N_CHUNKS // NW)
N_CHUNKS_PAD = CHUNKS_PER_TILE * NW

@functools.cache
def _sc_scatter_kernel():
    mesh = plsc.VectorSubcoreMesh(core_axis_name="c", subcore_axis_name="s",
                                  num_cores=NC, num_subcores=NS)
    return pl.kernel(
        _sc_scatter_body,
        out_type=(
            jax.ShapeDtypeStruct((NC, NPAD, L), jnp.float32),
            jax.ShapeDtypeStruct((NC, NPAD, L), jnp.float32),
        ),
        mesh=mesh,
        compiler_params=pltpu.CompilerParams(needs_layout_passes=False,
                                             use_tc_tiling_on_sc=False),
        scratch_types=[
            pltpu.VMEM((1, CHUNK), jnp.int32),
            pltpu.VMEM((CHUNK, L), jnp.float32),
            pltpu.VMEM((CHUNK, L), jnp.float32),
            pltpu.VMEM((STRIPE, L), jnp.float32),
            pltpu.VMEM_SHARED((NPAD, L), jnp.float32),
            pltpu.VMEM_SHARED((NPAD, L), jnp.float32),
        ],
    )


def _sc_scatter_body(colp_hbm, edge_hbm, sums_out, cnt_out,
                     idxbuf, databuf, onesbuf, zbuf, sums_sh, cnt_sh):
    c = lax.axis_index("c")
    s = lax.axis_index("s")
    w = s * NC + c
    z16 = jnp.zeros((L,), jnp.float32)
    ones16 = jnp.ones((L,), jnp.float32)

    def ones_body(n, carry):
        onesbuf[n, :] = ones16
        return carry
    lax.fori_loop(0, CHUNK, ones_body, 0)

    def zero_body(n, carry):
        zbuf[n, :] = z16
        return carry
    lax.fori_loop(0, STRIPE, zero_body, 0)
    pltpu.sync_copy(zbuf, sums_sh.at[pl.ds(s * STRIPE, STRIPE)])
    pltpu.sync_copy(zbuf, cnt_sh.at[pl.ds(s * STRIPE, STRIPE)])
    plsc.subcore_barrier()

    def chunk_body(j, carry):
        chunk = j * NW + w
        data_chunk = jnp.minimum(chunk, N_CHUNKS - 1)
        pltpu.sync_copy(colp_hbm.at[chunk], idxbuf.at[0])
        pltpu.sync_copy(edge_hbm.at[pl.ds(data_chunk * CHUNK, CHUNK)], databuf)
        pltpu.sync_copy(databuf, sums_sh.at[idxbuf.at[0]], add=True)
        pltpu.sync_copy(onesbuf, cnt_sh.at[idxbuf.at[0]], add=True)
        return carry
    lax.fori_loop(0, CHUNKS_PER_TILE, chunk_body, 0)
    plsc.subcore_barrier()

    pltpu.sync_copy(sums_sh.at[pl.ds(s * STRIPE, STRIPE)],
                    sums_out.at[c, pl.ds(s * STRIPE, STRIPE)])
    pltpu.sync_copy(cnt_sh.at[pl.ds(s * STRIPE, STRIPE)],
                    cnt_out.at[c, pl.ds(s * STRIPE, STRIPE)])


def _mlp_body(x_ref, sums_ref, cnt_ref, w1x_ref, w1a_ref, b1_ref, w2_ref,
              b2_ref, out_ref):
    x = x_ref[...]
    ssum = sums_ref[0] + sums_ref[1]
    cnt = cnt_ref[0] + cnt_ref[1]
    agg = ssum / jnp.maximum(cnt, 1.0)
    h = jnp.dot(x, w1x_ref[...], preferred_element_type=jnp.float32)
    h = h + jnp.dot(agg, w1a_ref[...], preferred_element_type=jnp.float32)
    h = jnp.maximum(h + b1_ref[...], 0.0)
    out_ref[...] = (x + jnp.dot(h, w2_ref[...],
                                preferred_element_type=jnp.float32)
                    + b2_ref[...])


_NB = 2000


def _tc_mlp(x, sums, cnt16, w1x, w1a, b1, w2, b2):
    return pl.pallas_call(
        _mlp_body,
        out_shape=jax.ShapeDtypeStruct((N_NODES, D_NODE), jnp.float32),
        grid=(N_NODES // _NB,),
        in_specs=[
            pl.BlockSpec((_NB, D_NODE), lambda i: (i, 0)),
            pl.BlockSpec((NC, _NB, L), lambda i: (0, i, 0)),
            pl.BlockSpec((NC, _NB, L), lambda i: (0, i, 0)),
            pl.BlockSpec((D_NODE, HID), lambda i: (0, 0)),
            pl.BlockSpec((L, HID), lambda i: (0, 0)),
            pl.BlockSpec((1, HID), lambda i: (0, 0)),
            pl.BlockSpec((HID, D_NODE), lambda i: (0, 0)),
            pl.BlockSpec((1, D_NODE), lambda i: (0, 0)),
        ],
        out_specs=pl.BlockSpec((_NB, D_NODE), lambda i: (i, 0)),
    )(x, sums, cnt16, w1x, w1a, b1, w2, b2)


@jax.jit
def kernel(x, edge_index, edge_attr, u, batch, W1, b1, W2, b2):
    col = edge_index[1]
    n_pad_idx = N_CHUNKS_PAD * CHUNK - N_EDGES
    pad_idx = N_NODES + (jnp.arange(n_pad_idx, dtype=jnp.int32)
                         % (NPAD - N_NODES))
    colp = jnp.concatenate([col, pad_idx]).reshape(N_CHUNKS_PAD, CHUNK)
    sums, cnt16 = _sc_scatter_kernel()(colp, edge_attr)
    return _tc_mlp(x, sums, cnt16, W1[:D_NODE], W1[D_NODE:],
                   b1.reshape(1, HID), W2, b2.reshape(1, D_NODE))


# speedup vs baseline: 4.8383x; 1.0215x over previous
import functools

import jax
import jax.numpy as jnp
from jax import lax
from jax.experimental import pallas as pl
from jax.experimental.pallas import tpu as pltpu
from jax.experimental.pallas import tpu_sc as plsc

NC = 2
NS = 16
L = 16
NW = NC * NS

N_NODES = 10000
N_EDGES = 320000
D_EDGE = 16
D_NODE = 128
HID = 64

NPAD = 10240
STRIPE = NPAD // NS
CHUNK = 128
N_CHUNKS = N_EDGES // CHUNK
CHUNKS_PER_TILE = -(-N_CHUNKS // NW)
N_CHUNKS_PAD = CHUNKS_PER_TILE * NW

@functools.cache
def _sc_scatter_kernel():
    mesh = plsc.VectorSubcoreMesh(core_axis_name="c", subcore_axis_name="s",
                                  num_cores=NC, num_subcores=NS)
    return pl.kernel(
        _sc_scatter_body,
        out_type=(
            jax.ShapeDtypeStruct((NC, NPAD, L), jnp.float32),
            jax.ShapeDtypeStruct((NC, NPAD, L), jnp.float32),
        ),
        mesh=mesh,
        compiler_params=pltpu.CompilerParams(needs_layout_passes=False,
                                             use_tc_tiling_on_sc=False),
        scratch_types=[
            pltpu.VMEM((1, CHUNK), jnp.int32),
            pltpu.VMEM((CHUNK, L), jnp.float32),
            pltpu.VMEM((CHUNK, L), jnp.float32),
            pltpu.VMEM((STRIPE, L), jnp.float32),
            pltpu.VMEM_SHARED((NPAD, L), jnp.float32),
            pltpu.VMEM_SHARED((NPAD, L), jnp.float32),
        ],
    )


def _sc_scatter_body(colp_hbm, edge_hbm, sums_out, cnt_out,
                     idxbuf, databuf, onesbuf, zbuf, sums_sh, cnt_sh):
    c = lax.axis_index("c")
    s = lax.axis_index("s")
    w = s * NC + c
    z16 = jnp.zeros((L,), jnp.float32)
    ones16 = jnp.ones((L,), jnp.float32)

    def ones_body(n, carry):
        onesbuf[n, :] = ones16
        return carry
    lax.fori_loop(0, CHUNK, ones_body, 0)

    def zero_body(n, carry):
        zbuf[n, :] = z16
        return carry
    lax.fori_loop(0, STRIPE, zero_body, 0)
    pltpu.sync_copy(zbuf, sums_sh.at[pl.ds(s * STRIPE, STRIPE)])
    pltpu.sync_copy(zbuf, cnt_sh.at[pl.ds(s * STRIPE, STRIPE)])
    plsc.subcore_barrier()

    def chunk_body(j, carry):
        chunk = j * NW + w
        data_chunk = jnp.minimum(chunk, N_CHUNKS - 1)
        pltpu.sync_copy(colp_hbm.at[chunk], idxbuf.at[0])
        pltpu.sync_copy(edge_hbm.at[pl.ds(data_chunk * CHUNK, CHUNK)], databuf)
        pltpu.sync_copy(databuf, sums_sh.at[idxbuf.at[0]], add=True)
        pltpu.sync_copy(onesbuf, cnt_sh.at[idxbuf.at[0]], add=True)
        return carry
    lax.fori_loop(0, CHUNKS_PER_TILE, chunk_body, 0)
    plsc.subcore_barrier()

    pltpu.sync_copy(sums_sh.at[pl.ds(s * STRIPE, STRIPE)],
                    sums_out.at[c, pl.ds(s * STRIPE, STRIPE)])
    pltpu.sync_copy(cnt_sh.at[pl.ds(s * STRIPE, STRIPE)],
                    cnt_out.at[c, pl.ds(s * STRIPE, STRIPE)])


_P = 8
_NBP = 256


def _mlp_body(xp_ref, sums_ref, cnt_ref, w1x_ref, w1a_ref, b1_ref, w2_ref,
              b2_ref, out_ref):
    xp = xp_ref[...]
    ssum = sums_ref[0] + sums_ref[1]
    cnt = cnt_ref[0] + cnt_ref[1]
    agg = ssum / jnp.maximum(cnt, 1.0)
    h = jnp.dot(xp, w1x_ref[...], preferred_element_type=jnp.float32)
    h = h + jnp.dot(agg, w1a_ref[...], preferred_element_type=jnp.float32)
    h = jnp.maximum(h + b1_ref[...], 0.0)
    out_ref[...] = (xp + jnp.dot(h, w2_ref[...],
                                 preferred_element_type=jnp.float32)
                    + b2_ref[...])


def _tc_mlp(xp, sums, cnt16, w1x_p, w1a_p, b1_p, w2_p, b2_p):
    np_rows = N_NODES // _P
    return pl.pallas_call(
        _mlp_body,
        out_shape=jax.ShapeDtypeStruct((np_rows, _P * D_NODE), jnp.float32),
        grid=(-(-np_rows // _NBP),),
        in_specs=[
            pl.BlockSpec((_NBP, _P * D_NODE), lambda i: (i, 0)),
            pl.BlockSpec((NC, _NBP, _P * L), lambda i: (0, i, 0)),
            pl.BlockSpec((NC, _NBP, _P * L), lambda i: (0, i, 0)),
            pl.BlockSpec((_P * D_NODE, _P * HID), lambda i: (0, 0)),
            pl.BlockSpec((_P * L, _P * HID), lambda i: (0, 0)),
            pl.BlockSpec((1, _P * HID), lambda i: (0, 0)),
            pl.BlockSpec((_P * HID, _P * D_NODE), lambda i: (0, 0)),
            pl.BlockSpec((1, _P * D_NODE), lambda i: (0, 0)),
        ],
        out_specs=pl.BlockSpec((_NBP, _P * D_NODE), lambda i: (i, 0)),
    )(xp, sums, cnt16, w1x_p, w1a_p, b1_p, w2_p, b2_p)


@jax.jit
def kernel(x, edge_index, edge_attr, u, batch, W1, b1, W2, b2):
    col = edge_index[1]
    n_pad_idx = N_CHUNKS_PAD * CHUNK - N_EDGES
    pad_idx = N_NODES + (jnp.arange(n_pad_idx, dtype=jnp.int32)
                         % (NPAD - N_NODES))
    colp = jnp.concatenate([col, pad_idx]).reshape(N_CHUNKS_PAD, CHUNK)
    sums, cnt16 = _sc_scatter_kernel()(colp, edge_attr)
    sums = sums.reshape(NC, NPAD // _P, _P * L)
    cnt16 = cnt16.reshape(NC, NPAD // _P, _P * L)
    xp = x.reshape(N_NODES // _P, _P * D_NODE)
    eye = jnp.eye(_P, dtype=jnp.float32)
    w1x_p = jnp.kron(eye, W1[:D_NODE])
    w1a_p = jnp.kron(eye, W1[D_NODE:])
    w2_p = jnp.kron(eye, W2)
    b1_p = jnp.tile(b1, _P).reshape(1, _P * HID)
    b2_p = jnp.tile(b2, _P).reshape(1, _P * D_NODE)
    outp = _tc_mlp(xp, sums, cnt16, w1x_p, w1a_p, b1_p, w2_p, b2_p)
    return outp.reshape(N_NODES, D_NODE)


# speedup vs baseline: 6.5586x; 1.3556x over previous
import functools

import jax
import jax.numpy as jnp
from jax import lax
from jax.experimental import pallas as pl
from jax.experimental.pallas import tpu as pltpu
from jax.experimental.pallas import tpu_sc as plsc

NC = 2
NS = 16
L = 16
NW = NC * NS

N_NODES = 10000
N_EDGES = 320000
D_EDGE = 16
D_NODE = 128
HID = 64

NPAD = 10240
STRIPE = NPAD // NS
CHUNK = 128
N_CHUNKS = N_EDGES // CHUNK
CHUNKS_PER_TILE = 80
N_CHUNKS_PAD = CHUNKS_PER_TILE * NW


@functools.cache
def _sc_scatter_kernel():
    mesh = plsc.VectorSubcoreMesh(core_axis_name="c", subcore_axis_name="s",
                                  num_cores=NC, num_subcores=NS)
    return pl.kernel(
        _sc_scatter_body,
        out_type=(
            jax.ShapeDtypeStruct((NC, NPAD, L), jnp.float32),
            jax.ShapeDtypeStruct((NC, NPAD, L), jnp.float32),
        ),
        mesh=mesh,
        compiler_params=pltpu.CompilerParams(needs_layout_passes=False,
                                             use_tc_tiling_on_sc=False),
        scratch_types=[
            pltpu.VMEM((CHUNKS_PER_TILE, CHUNK), jnp.int32),
            pltpu.VMEM((CHUNK, L), jnp.float32),
            pltpu.VMEM((CHUNK, L), jnp.float32),
            pltpu.VMEM((CHUNK, L), jnp.float32),
            pltpu.VMEM((STRIPE, L), jnp.float32),
            pltpu.VMEM_SHARED((NPAD, L), jnp.float32),
            pltpu.VMEM_SHARED((NPAD, L), jnp.float32),
            pltpu.SemaphoreType.DMA,
            pltpu.SemaphoreType.DMA,
            pltpu.SemaphoreType.DMA,
            pltpu.SemaphoreType.DMA,
            pltpu.SemaphoreType.DMA,
            pltpu.SemaphoreType.DMA,
        ],
    )


def _sc_scatter_body(colp_hbm, edge_hbm, sums_out, cnt_out,
                     idxall, dbuf0, dbuf1, onesbuf, zbuf, sums_sh, cnt_sh,
                     dsem0, dsem1, ssem0, ssem1, osem0, osem1):
    c = lax.axis_index("c")
    s = lax.axis_index("s")
    w = s * NC + c
    z16 = jnp.zeros((L,), jnp.float32)
    ones16 = jnp.ones((L,), jnp.float32)
    dbufs = (dbuf0, dbuf1)
    dsems = (dsem0, dsem1)
    ssems = (ssem0, ssem1)
    osems = (osem0, osem1)

    def ones_body(n, carry):
        onesbuf[n, :] = ones16
        return carry
    lax.fori_loop(0, CHUNK, ones_body, 0)

    def zero_body(n, carry):
        zbuf[n, :] = z16
        return carry
    lax.fori_loop(0, STRIPE, zero_body, 0)
    pltpu.sync_copy(zbuf, sums_sh.at[pl.ds(s * STRIPE, STRIPE)])
    pltpu.sync_copy(zbuf, cnt_sh.at[pl.ds(s * STRIPE, STRIPE)])

    pltpu.sync_copy(colp_hbm.at[w], idxall)
    plsc.subcore_barrier()

    def _data_chunk(j):
        return jnp.minimum(j * NW + w, N_CHUNKS - 1) * CHUNK

    pltpu.async_copy(edge_hbm.at[pl.ds(_data_chunk(0), CHUNK)], dbuf0, dsem0)
    pltpu.async_copy(edge_hbm.at[pl.ds(_data_chunk(1), CHUNK)], dbuf1, dsem1)

    def pair_body(t, carry):
        for b in range(2):
            j = 2 * t + b
            dbuf = dbufs[b]
            pltpu.make_async_copy(edge_hbm.at[pl.ds(0, CHUNK)], dbuf,
                                  dsems[b]).wait()
            sc = pltpu.async_copy(dbuf, sums_sh.at[idxall.at[j]], ssems[b],
                                  add=True)
            oc = pltpu.async_copy(onesbuf, cnt_sh.at[idxall.at[j]], osems[b],
                                  add=True)
            sc.wait()
            oc.wait()

            @pl.when(j + 2 < CHUNKS_PER_TILE)
            def _():
                pltpu.async_copy(
                    edge_hbm.at[pl.ds(_data_chunk(j + 2), CHUNK)], dbuf,
                    dsems[b])
        return carry
    lax.fori_loop(0, CHUNKS_PER_TILE // 2, pair_body, 0)
    plsc.subcore_barrier()

    pltpu.sync_copy(sums_sh.at[pl.ds(s * STRIPE, STRIPE)],
                    sums_out.at[c, pl.ds(s * STRIPE, STRIPE)])
    pltpu.sync_copy(cnt_sh.at[pl.ds(s * STRIPE, STRIPE)],
                    cnt_out.at[c, pl.ds(s * STRIPE, STRIPE)])


_P = 8
_NBP = 256


def _mlp_body(xp_ref, sums_ref, cnt_ref, w1x_ref, w1a_ref, b1_ref, w2_ref,
              b2_ref, out_ref):
    xp = xp_ref[...]
    ssum = sums_ref[0] + sums_ref[1]
    cnt = cnt_ref[0] + cnt_ref[1]
    agg = ssum / jnp.maximum(cnt, 1.0)
    h = jnp.dot(xp, w1x_ref[...], preferred_element_type=jnp.float32)
    h = h + jnp.dot(agg, w1a_ref[...], preferred_element_type=jnp.float32)
    h = jnp.maximum(h + b1_ref[...], 0.0)
    out_ref[...] = (xp + jnp.dot(h, w2_ref[...],
                                 preferred_element_type=jnp.float32)
                    + b2_ref[...])


def _tc_mlp(xp, sums, cnt16, w1x_p, w1a_p, b1_p, w2_p, b2_p):
    np_rows = N_NODES // _P
    return pl.pallas_call(
        _mlp_body,
        out_shape=jax.ShapeDtypeStruct((np_rows, _P * D_NODE), jnp.float32),
        grid=(-(-np_rows // _NBP),),
        in_specs=[
            pl.BlockSpec((_NBP, _P * D_NODE), lambda i: (i, 0)),
            pl.BlockSpec((NC, _NBP, _P * L), lambda i: (0, i, 0)),
            pl.BlockSpec((NC, _NBP, _P * L), lambda i: (0, i, 0)),
            pl.BlockSpec((_P * D_NODE, _P * HID), lambda i: (0, 0)),
            pl.BlockSpec((_P * L, _P * HID), lambda i: (0, 0)),
            pl.BlockSpec((1, _P * HID), lambda i: (0, 0)),
            pl.BlockSpec((_P * HID, _P * D_NODE), lambda i: (0, 0)),
            pl.BlockSpec((1, _P * D_NODE), lambda i: (0, 0)),
        ],
        out_specs=pl.BlockSpec((_NBP, _P * D_NODE), lambda i: (i, 0)),
    )(xp, sums, cnt16, w1x_p, w1a_p, b1_p, w2_p, b2_p)


@jax.jit
def kernel(x, edge_index, edge_attr, u, batch, W1, b1, W2, b2):
    col = edge_index[1]
    n_pad_idx = N_CHUNKS_PAD * CHUNK - N_EDGES
    pad_idx = N_NODES + (jnp.arange(n_pad_idx, dtype=jnp.int32)
                         % (NPAD - N_NODES))
    colp3 = (jnp.concatenate([col, pad_idx])
             .reshape(CHUNKS_PER_TILE, NW, CHUNK).transpose(1, 0, 2))
    sums, cnt16 = _sc_scatter_kernel()(colp3, edge_attr)
    sums = sums.reshape(NC, NPAD // _P, _P * L)
    cnt16 = cnt16.reshape(NC, NPAD // _P, _P * L)
    xp = x.reshape(N_NODES // _P, _P * D_NODE)
    eye = jnp.eye(_P, dtype=jnp.float32)
    w1x_p = jnp.kron(eye, W1[:D_NODE])
    w1a_p = jnp.kron(eye, W1[D_NODE:])
    w2_p = jnp.kron(eye, W2)
    b1_p = jnp.tile(b1, _P).reshape(1, _P * HID)
    b2_p = jnp.tile(b2, _P).reshape(1, _P * D_NODE)
    outp = _tc_mlp(xp, sums, cnt16, w1x_p, w1a_p, b1_p, w2_p, b2_p)
    return outp.reshape(N_NODES, D_NODE)


# speedup vs baseline: 11.4347x; 1.7435x over previous
import functools

import jax
import jax.numpy as jnp
from jax import lax
from jax.experimental import pallas as pl
from jax.experimental.pallas import tpu as pltpu
from jax.experimental.pallas import tpu_sc as plsc

NC = 2
NS = 16
L = 16
NW = NC * NS

N_NODES = 10000
N_EDGES = 320000
D_EDGE = 16
D_NODE = 128
HID = 64

NPAD = 10240
E_HALF = N_EDGES // NC
EC = 16000
N_EC = E_HALF // EC
E_CNT = N_EDGES // NW


@functools.cache
def _sc_scatter_kernel():
    mesh = plsc.VectorSubcoreMesh(core_axis_name="c", subcore_axis_name="s",
                                  num_cores=NC, num_subcores=NS)
    return pl.kernel(
        _sc_scatter_body,
        out_type=(
            jax.ShapeDtypeStruct((NC, NS, NPAD), jnp.float32),
            jax.ShapeDtypeStruct((NC, NS, NPAD), jnp.float32),
        ),
        mesh=mesh,
        compiler_params=pltpu.CompilerParams(needs_layout_passes=False),
        scratch_types=[
            pltpu.VMEM((EC,), jnp.float32),
            pltpu.VMEM((EC,), jnp.float32),
            pltpu.VMEM((EC,), jnp.int32),
            pltpu.VMEM((EC,), jnp.int32),
            pltpu.VMEM((E_CNT,), jnp.int32),
            pltpu.VMEM((NPAD,), jnp.float32),
            pltpu.VMEM((NPAD,), jnp.float32),
            pltpu.SemaphoreType.DMA,
            pltpu.SemaphoreType.DMA,
            pltpu.SemaphoreType.DMA,
            pltpu.SemaphoreType.DMA,
        ],
    )


def _sc_scatter_body(eaT_hbm, col_hbm, sums_out, cnt_out,
                     vbuf0, vbuf1, ibuf0, ibuf1, cbuf, acc_s, acc_c,
                     vsem0, vsem1, isem0, isem1):
    c = lax.axis_index("c")
    f = lax.axis_index("s")
    z16 = jnp.zeros((L,), jnp.float32)
    ones16 = jnp.ones((L,), jnp.float32)
    vbufs = (vbuf0, vbuf1)
    ibufs = (ibuf0, ibuf1)
    vsems = (vsem0, vsem1)
    isems = (isem0, isem1)
    ebase = c * E_HALF

    def zero_body(n, carry):
        acc_s[pl.ds(n * L, L)] = z16
        acc_c[pl.ds(n * L, L)] = z16
        return carry
    lax.fori_loop(0, NPAD // L, zero_body, 0)

    def _start(k, b):
        pltpu.async_copy(eaT_hbm.at[f, pl.ds(ebase + k * EC, EC)],
                         vbufs[b], vsems[b])
        pltpu.async_copy(col_hbm.at[pl.ds(ebase + k * EC, EC)],
                         ibufs[b], isems[b])
    _start(0, 0)
    _start(1, 1)

    w = f * NC + c
    pltpu.sync_copy(col_hbm.at[pl.ds(w * E_CNT, E_CNT)], cbuf)

    def cnt_body(i, carry):
        iv = cbuf[pl.ds(i * L, L)]
        plsc.addupdate_scatter(acc_c, [iv], ones16)
        return carry
    lax.fori_loop(0, E_CNT // L, cnt_body, 0)

    def pair_body(t, carry):
        for b in range(2):
            k = 2 * t + b
            pltpu.make_async_copy(eaT_hbm.at[f, pl.ds(0, EC)], vbufs[b],
                                  vsems[b]).wait()
            pltpu.make_async_copy(col_hbm.at[pl.ds(0, EC)], ibufs[b],
                                  isems[b]).wait()

            def scat_body(i, carry2):
                iv = ibufs[b][pl.ds(i * L, L)]
                vv = vbufs[b][pl.ds(i * L, L)]
                plsc.addupdate_scatter(acc_s, [iv], vv)
                return carry2
            lax.fori_loop(0, EC // L, scat_body, 0)

            @pl.when(k + 2 < N_EC)
            def _():
                _start(k + 2, b)
        return carry
    lax.fori_loop(0, N_EC // 2, pair_body, 0)

    pltpu.sync_copy(acc_s, sums_out.at[c, f])
    pltpu.sync_copy(acc_c, cnt_out.at[c, f])


_NB = 2048


def _mlp_body(x_ref, sums_ref, cnt_ref, w1x_ref, w1a_ref, b1_ref, w2_ref,
              b2_ref, out_ref):
    x = x_ref[...]
    s_t = sums_ref[0] + sums_ref[1]
    cnt = jnp.sum(cnt_ref[0] + cnt_ref[1], axis=0)
    agg_t = s_t / jnp.maximum(cnt, 1.0)[None, :]
    h = jnp.dot(x, w1x_ref[...], preferred_element_type=jnp.float32)
    h = h + lax.dot_general(agg_t, w1a_ref[...], (((0,), (0,)), ((), ())),
                            preferred_element_type=jnp.float32)
    h = jnp.maximum(h + b1_ref[...], 0.0)
    out_ref[...] = (x + jnp.dot(h, w2_ref[...],
                                preferred_element_type=jnp.float32)
                    + b2_ref[...])


def _tc_mlp(x, sums_t, cnt_p, w1x, w1a, b1, w2, b2):
    return pl.pallas_call(
        _mlp_body,
        out_shape=jax.ShapeDtypeStruct((N_NODES, D_NODE), jnp.float32),
        grid=(-(-N_NODES // _NB),),
        in_specs=[
            pl.BlockSpec((_NB, D_NODE), lambda i: (i, 0)),
            pl.BlockSpec((NC, NS, _NB), lambda i: (0, 0, i)),
            pl.BlockSpec((NC, NS, _NB), lambda i: (0, 0, i)),
            pl.BlockSpec((D_NODE, HID), lambda i: (0, 0)),
            pl.BlockSpec((L, HID), lambda i: (0, 0)),
            pl.BlockSpec((1, HID), lambda i: (0, 0)),
            pl.BlockSpec((HID, D_NODE), lambda i: (0, 0)),
            pl.BlockSpec((1, D_NODE), lambda i: (0, 0)),
        ],
        out_specs=pl.BlockSpec((_NB, D_NODE), lambda i: (i, 0)),
    )(x, sums_t, cnt_p, w1x, w1a, b1, w2, b2)


@jax.jit
def kernel(x, edge_index, edge_attr, u, batch, W1, b1, W2, b2):
    col = edge_index[1]
    ea_t = edge_attr.T
    sums_t, cnt_p = _sc_scatter_kernel()(ea_t, col)
    return _tc_mlp(x, sums_t, cnt_p, W1[:D_NODE], W1[D_NODE:],
                   b1.reshape(1, HID), W2, b2.reshape(1, D_NODE))


# speedup vs baseline: 11.6829x; 1.0217x over previous
import functools

import jax
import jax.numpy as jnp
from jax import lax
from jax.experimental import pallas as pl
from jax.experimental.pallas import tpu as pltpu
from jax.experimental.pallas import tpu_sc as plsc

NC = 2
NS = 16
L = 16
NW = NC * NS

N_NODES = 10000
N_EDGES = 320000
D_EDGE = 16
D_NODE = 128
HID = 64

NPAD = 10240
E_HALF = N_EDGES // NC
EC = 16000
N_EC = E_HALF // EC
E_CNT = N_EDGES // NW
_UNROLL = 8


@functools.cache
def _sc_scatter_kernel():
    mesh = plsc.VectorSubcoreMesh(core_axis_name="c", subcore_axis_name="s",
                                  num_cores=NC, num_subcores=NS)
    return pl.kernel(
        _sc_scatter_body,
        out_type=(
            jax.ShapeDtypeStruct((NC, NS, NPAD), jnp.float32),
            jax.ShapeDtypeStruct((NC, NS, NPAD), jnp.float32),
        ),
        mesh=mesh,
        compiler_params=pltpu.CompilerParams(needs_layout_passes=False),
        scratch_types=[
            pltpu.VMEM((EC,), jnp.float32),
            pltpu.VMEM((EC,), jnp.float32),
            pltpu.VMEM((EC,), jnp.int32),
            pltpu.VMEM((EC,), jnp.int32),
            pltpu.VMEM((E_CNT,), jnp.int32),
            pltpu.VMEM((NPAD,), jnp.float32),
            pltpu.VMEM((NPAD,), jnp.float32),
            pltpu.SemaphoreType.DMA,
            pltpu.SemaphoreType.DMA,
            pltpu.SemaphoreType.DMA,
            pltpu.SemaphoreType.DMA,
        ],
    )


def _sc_scatter_body(eaT_hbm, col_hbm, sums_out, cnt_out,
                     vbuf0, vbuf1, ibuf0, ibuf1, cbuf, acc_s, acc_c,
                     vsem0, vsem1, isem0, isem1):
    c = lax.axis_index("c")
    f = lax.axis_index("s")
    z16 = jnp.zeros((L,), jnp.float32)
    ones16 = jnp.ones((L,), jnp.float32)
    vbufs = (vbuf0, vbuf1)
    ibufs = (ibuf0, ibuf1)
    vsems = (vsem0, vsem1)
    isems = (isem0, isem1)
    ebase = c * E_HALF

    def zero_body(n, carry):
        acc_s[pl.ds(n * L, L)] = z16
        acc_c[pl.ds(n * L, L)] = z16
        return carry
    lax.fori_loop(0, NPAD // L, zero_body, 0)

    def _start(k, b):
        pltpu.async_copy(eaT_hbm.at[f, pl.ds(ebase + k * EC, EC)],
                         vbufs[b], vsems[b])
        pltpu.async_copy(col_hbm.at[pl.ds(ebase + k * EC, EC)],
                         ibufs[b], isems[b])
    _start(0, 0)
    _start(1, 1)

    w = f * NC + c
    pltpu.sync_copy(col_hbm.at[pl.ds(w * E_CNT, E_CNT)], cbuf)

    def cnt_body(i, carry):
        for u in range(5):
            iv = cbuf[pl.ds((i * 5 + u) * L, L)]
            plsc.addupdate_scatter(acc_c, [iv], ones16)
        return carry
    lax.fori_loop(0, E_CNT // L // 5, cnt_body, 0)

    def pair_body(t, carry):
        for b in range(2):
            k = 2 * t + b
            pltpu.make_async_copy(eaT_hbm.at[f, pl.ds(0, EC)], vbufs[b],
                                  vsems[b]).wait()
            pltpu.make_async_copy(col_hbm.at[pl.ds(0, EC)], ibufs[b],
                                  isems[b]).wait()

            def scat_body(i, carry2):
                for u in range(_UNROLL):
                    iv = ibufs[b][pl.ds((i * _UNROLL + u) * L, L)]
                    vv = vbufs[b][pl.ds((i * _UNROLL + u) * L, L)]
                    plsc.addupdate_scatter(acc_s, [iv], vv)
                return carry2
            lax.fori_loop(0, EC // L // _UNROLL, scat_body, 0)

            @pl.when(k + 2 < N_EC)
            def _():
                _start(k + 2, b)
        return carry
    lax.fori_loop(0, N_EC // 2, pair_body, 0)

    pltpu.sync_copy(acc_s, sums_out.at[c, f])
    pltpu.sync_copy(acc_c, cnt_out.at[c, f])


_NB = 2048


def _mlp_body(x_ref, sums_ref, cnt_ref, w1x_ref, w1a_ref, b1_ref, w2_ref,
              b2_ref, out_ref):
    x = x_ref[...]
    s_t = sums_ref[0] + sums_ref[1]
    cnt = jnp.sum(cnt_ref[0] + cnt_ref[1], axis=0)
    agg_t = s_t / jnp.maximum(cnt, 1.0)[None, :]
    h = jnp.dot(x, w1x_ref[...], preferred_element_type=jnp.float32)
    h = h + lax.dot_general(agg_t, w1a_ref[...], (((0,), (0,)), ((), ())),
                            preferred_element_type=jnp.float32)
    h = jnp.maximum(h + b1_ref[...], 0.0)
    out_ref[...] = (x + jnp.dot(h, w2_ref[...],
                                preferred_element_type=jnp.float32)
                    + b2_ref[...])


def _tc_mlp(x, sums_t, cnt_p, w1x, w1a, b1, w2, b2):
    return pl.pallas_call(
        _mlp_body,
        out_shape=jax.ShapeDtypeStruct((N_NODES, D_NODE), jnp.float32),
        grid=(-(-N_NODES // _NB),),
        in_specs=[
            pl.BlockSpec((_NB, D_NODE), lambda i: (i, 0)),
            pl.BlockSpec((NC, NS, _NB), lambda i: (0, 0, i)),
            pl.BlockSpec((NC, NS, _NB), lambda i: (0, 0, i)),
            pl.BlockSpec((D_NODE, HID), lambda i: (0, 0)),
            pl.BlockSpec((L, HID), lambda i: (0, 0)),
            pl.BlockSpec((1, HID), lambda i: (0, 0)),
            pl.BlockSpec((HID, D_NODE), lambda i: (0, 0)),
            pl.BlockSpec((1, D_NODE), lambda i: (0, 0)),
        ],
        out_specs=pl.BlockSpec((_NB, D_NODE), lambda i: (i, 0)),
    )(x, sums_t, cnt_p, w1x, w1a, b1, w2, b2)


@jax.jit
def kernel(x, edge_index, edge_attr, u, batch, W1, b1, W2, b2):
    col = edge_index[1]
    ea_t = edge_attr.T
    sums_t, cnt_p = _sc_scatter_kernel()(ea_t, col)
    return _tc_mlp(x, sums_t, cnt_p, W1[:D_NODE], W1[D_NODE:],
                   b1.reshape(1, HID), W2, b2.reshape(1, D_NODE))


# speedup vs baseline: 13.3652x; 1.1440x over previous
import functools

import jax
import jax.numpy as jnp
from jax import lax
from jax.experimental import pallas as pl
from jax.experimental.pallas import tpu as pltpu
from jax.experimental.pallas import tpu_sc as plsc

NC = 2
NS = 16
L = 16
NW = NC * NS

N_NODES = 10000
N_EDGES = 320000
D_EDGE = 16
D_NODE = 128
HID = 64

NPAD = 10240
E_HALF = N_EDGES // NC
EC = 3200
N_EC = E_HALF // EC
CNT_W = 10112
_UNROLL = 8


@functools.cache
def _sc_scatter_kernel():
    mesh = plsc.VectorSubcoreMesh(core_axis_name="c", subcore_axis_name="s",
                                  num_cores=NC, num_subcores=NS)
    return pl.kernel(
        _sc_scatter_body,
        out_type=(
            jax.ShapeDtypeStruct((NC, NS, NPAD), jnp.float32),
            jax.ShapeDtypeStruct((NC, NS, NPAD), jnp.float32),
        ),
        mesh=mesh,
        compiler_params=pltpu.CompilerParams(needs_layout_passes=False),
        scratch_types=[
            pltpu.VMEM((EC,), jnp.float32),
            pltpu.VMEM((EC,), jnp.float32),
            pltpu.VMEM((2, EC), jnp.int32),
            pltpu.VMEM((2, EC), jnp.int32),
            pltpu.VMEM((2, CNT_W), jnp.int32),
            pltpu.VMEM((NPAD,), jnp.float32),
            pltpu.VMEM((NPAD,), jnp.float32),
            pltpu.SemaphoreType.DMA,
            pltpu.SemaphoreType.DMA,
            pltpu.SemaphoreType.DMA,
            pltpu.SemaphoreType.DMA,
        ],
    )


def _sc_scatter_body(eaT_hbm, col_hbm, sums_out, cnt_out,
                     vbuf0, vbuf1, ibuf0, ibuf1, cbuf, acc_s, acc_c,
                     vsem0, vsem1, isem0, isem1):
    c = lax.axis_index("c")
    f = lax.axis_index("s")
    z16 = jnp.zeros((L,), jnp.float32)
    ones16 = jnp.ones((L,), jnp.float32)
    vbufs = (vbuf0, vbuf1)
    ibufs = (ibuf0, ibuf1)
    vsems = (vsem0, vsem1)
    isems = (isem0, isem1)
    ebase = c * E_HALF

    def zero_body(n, carry):
        acc_s[pl.ds(n * L, L)] = z16
        acc_c[pl.ds(n * L, L)] = z16
        return carry
    lax.fori_loop(0, NPAD // L, zero_body, 0)

    def _start(k, b):
        pltpu.async_copy(eaT_hbm.at[f, pl.ds(ebase + k * EC, EC)],
                         vbufs[b], vsems[b])
        pltpu.async_copy(col_hbm.at[:, pl.ds(ebase + k * EC, EC)],
                         ibufs[b], isems[b])
    _start(0, 0)
    _start(1, 1)

    w = f * NC + c
    nch = 78 + (w >= 28).astype(jnp.int32)
    coff = (78 * w + jnp.maximum(w - 28, 0)) * 128
    pltpu.sync_copy(col_hbm.at[:, pl.ds(coff, CNT_W)], cbuf)

    def cnt_body(i, carry):
        for u in range(8):
            iv = cbuf[1, pl.ds((i * 8 + u) * L, L)]
            plsc.addupdate_scatter(acc_c, [iv], ones16)
        return carry
    lax.fori_loop(0, nch, cnt_body, 0)

    def pair_body(t, carry):
        for b in range(2):
            k = 2 * t + b
            pltpu.make_async_copy(eaT_hbm.at[f, pl.ds(0, EC)], vbufs[b],
                                  vsems[b]).wait()
            pltpu.make_async_copy(col_hbm.at[:, pl.ds(0, EC)], ibufs[b],
                                  isems[b]).wait()

            def scat_body(i, carry2):
                for u in range(_UNROLL):
                    iv = ibufs[b][1, pl.ds((i * _UNROLL + u) * L, L)]
                    vv = vbufs[b][pl.ds((i * _UNROLL + u) * L, L)]
                    plsc.addupdate_scatter(acc_s, [iv], vv)
                return carry2
            lax.fori_loop(0, EC // L // _UNROLL, scat_body, 0)

            @pl.when(k + 2 < N_EC)
            def _():
                _start(k + 2, b)
        return carry
    lax.fori_loop(0, N_EC // 2, pair_body, 0)

    pltpu.sync_copy(acc_s, sums_out.at[c, f])
    pltpu.sync_copy(acc_c, cnt_out.at[c, f])


_NB = 2048


def _mlp_body(x_ref, sums_ref, cnt_ref, w1x_ref, w1a_ref, b1_ref, w2_ref,
              b2_ref, out_ref):
    x = x_ref[...]
    s_t = sums_ref[0] + sums_ref[1]
    cnt = jnp.sum(cnt_ref[0] + cnt_ref[1], axis=0)
    agg_t = s_t / jnp.maximum(cnt, 1.0)[None, :]
    h = jnp.dot(x, w1x_ref[...], preferred_element_type=jnp.float32)
    h = h + lax.dot_general(agg_t, w1a_ref[...], (((0,), (0,)), ((), ())),
                            preferred_element_type=jnp.float32)
    h = jnp.maximum(h + b1_ref[...], 0.0)
    out_ref[...] = (x + jnp.dot(h, w2_ref[...],
                                preferred_element_type=jnp.float32)
                    + b2_ref[...])


def _tc_mlp(x, sums_t, cnt_p, w1x, w1a, b1, w2, b2):
    return pl.pallas_call(
        _mlp_body,
        out_shape=jax.ShapeDtypeStruct((N_NODES, D_NODE), jnp.float32),
        grid=(-(-N_NODES // _NB),),
        in_specs=[
            pl.BlockSpec((_NB, D_NODE), lambda i: (i, 0)),
            pl.BlockSpec((NC, NS, _NB), lambda i: (0, 0, i)),
            pl.BlockSpec((NC, NS, _NB), lambda i: (0, 0, i)),
            pl.BlockSpec((D_NODE, HID), lambda i: (0, 0)),
            pl.BlockSpec((L, HID), lambda i: (0, 0)),
            pl.BlockSpec((1, HID), lambda i: (0, 0)),
            pl.BlockSpec((HID, D_NODE), lambda i: (0, 0)),
            pl.BlockSpec((1, D_NODE), lambda i: (0, 0)),
        ],
        out_specs=pl.BlockSpec((_NB, D_NODE), lambda i: (i, 0)),
    )(x, sums_t, cnt_p, w1x, w1a, b1, w2, b2)


@jax.jit
def kernel(x, edge_index, edge_attr, u, batch, W1, b1, W2, b2):
    ea_t = edge_attr.T
    sums_t, cnt_p = _sc_scatter_kernel()(ea_t, edge_index)
    return _tc_mlp(x, sums_t, cnt_p, W1[:D_NODE], W1[D_NODE:],
                   b1.reshape(1, HID), W2, b2.reshape(1, D_NODE))


# speedup vs baseline: 17.2988x; 1.2943x over previous
import functools

import jax
import jax.numpy as jnp
from jax import lax
from jax.experimental import pallas as pl
from jax.experimental.pallas import tpu as pltpu
from jax.experimental.pallas import tpu_sc as plsc

NC = 2
NS = 16
L = 16
NW = NC * NS

N_NODES = 10000
N_EDGES = 320000
D_EDGE = 16
D_NODE = 128
HID = 64

NPAD = 10240
E_HALF = N_EDGES // NC
EC = 3200
N_EC = E_HALF // EC
CNT_W = 10112
_UNROLL = 8


@functools.cache
def _sc_scatter_kernel():
    mesh = plsc.VectorSubcoreMesh(core_axis_name="c", subcore_axis_name="s",
                                  num_cores=NC, num_subcores=NS)
    return pl.kernel(
        _sc_scatter_body,
        out_type=(
            jax.ShapeDtypeStruct((NC, NS, NPAD), jnp.float32),
            jax.ShapeDtypeStruct((NC, NS, NPAD), jnp.float32),
        ),
        mesh=mesh,
        compiler_params=pltpu.CompilerParams(needs_layout_passes=False),
        scratch_types=[
            pltpu.VMEM((EC,), jnp.float32),
            pltpu.VMEM((EC,), jnp.float32),
            pltpu.VMEM((2, EC), jnp.int32),
            pltpu.VMEM((2, EC), jnp.int32),
            pltpu.VMEM((2, CNT_W), jnp.int32),
            pltpu.VMEM((NPAD,), jnp.float32),
            pltpu.VMEM((NPAD,), jnp.float32),
            pltpu.SemaphoreType.DMA,
            pltpu.SemaphoreType.DMA,
            pltpu.SemaphoreType.DMA,
            pltpu.SemaphoreType.DMA,
        ],
    )


def _sc_scatter_body(eaT_hbm, col_hbm, sums_out, cnt_out,
                     vbuf0, vbuf1, ibuf0, ibuf1, cbuf, acc_s, acc_c,
                     vsem0, vsem1, isem0, isem1):
    c = lax.axis_index("c")
    f = lax.axis_index("s")
    z16 = jnp.zeros((L,), jnp.float32)
    ones16 = jnp.ones((L,), jnp.float32)
    vbufs = (vbuf0, vbuf1)
    ibufs = (ibuf0, ibuf1)
    vsems = (vsem0, vsem1)
    isems = (isem0, isem1)
    ebase = c * E_HALF

    def zero_body(n, carry):
        acc_s[pl.ds(n * L, L)] = z16
        acc_c[pl.ds(n * L, L)] = z16
        return carry
    lax.fori_loop(0, NPAD // L, zero_body, 0)

    def _start(k, b):
        pltpu.async_copy(eaT_hbm.at[f, pl.ds(ebase + k * EC, EC)],
                         vbufs[b], vsems[b])
        pltpu.async_copy(col_hbm.at[:, pl.ds(ebase + k * EC, EC)],
                         ibufs[b], isems[b])
    _start(0, 0)
    _start(1, 1)

    w = f * NC + c
    nch = 78 + (w >= 28).astype(jnp.int32)
    coff = (78 * w + jnp.maximum(w - 28, 0)) * 128
    pltpu.sync_copy(col_hbm.at[:, pl.ds(coff, CNT_W)], cbuf)

    def cnt_body(i, carry):
        ivs = [cbuf[1, pl.ds((i * 8 + u) * L, L)] for u in range(8)]
        for u in range(8):
            plsc.addupdate_scatter(acc_c, [ivs[u]], ones16)
        return carry
    lax.fori_loop(0, nch, cnt_body, 0)

    def pair_body(t, carry):
        for b in range(2):
            k = 2 * t + b
            pltpu.make_async_copy(eaT_hbm.at[f, pl.ds(0, EC)], vbufs[b],
                                  vsems[b]).wait()
            pltpu.make_async_copy(col_hbm.at[:, pl.ds(0, EC)], ibufs[b],
                                  isems[b]).wait()

            def scat_body(i, carry2):
                ivs = [ibufs[b][1, pl.ds((i * _UNROLL + u) * L, L)]
                       for u in range(_UNROLL)]
                vvs = [vbufs[b][pl.ds((i * _UNROLL + u) * L, L)]
                       for u in range(_UNROLL)]
                for u in range(_UNROLL):
                    plsc.addupdate_scatter(acc_s, [ivs[u]], vvs[u])
                return carry2
            lax.fori_loop(0, EC // L // _UNROLL, scat_body, 0)

            @pl.when(k + 2 < N_EC)
            def _():
                _start(k + 2, b)
        return carry
    lax.fori_loop(0, N_EC // 2, pair_body, 0)

    pltpu.sync_copy(acc_s, sums_out.at[c, f])
    pltpu.sync_copy(acc_c, cnt_out.at[c, f])


_NB = 2048


def _mlp_body(x_ref, sums_ref, cnt_ref, w1x_ref, w1a_ref, b1_ref, w2_ref,
              b2_ref, out_ref):
    x = x_ref[...]
    s_t = sums_ref[0] + sums_ref[1]
    cnt = jnp.sum(cnt_ref[0] + cnt_ref[1], axis=0)
    agg_t = s_t / jnp.maximum(cnt, 1.0)[None, :]
    h = jnp.dot(x, w1x_ref[...], preferred_element_type=jnp.float32)
    h = h + lax.dot_general(agg_t, w1a_ref[...], (((0,), (0,)), ((), ())),
                            preferred_element_type=jnp.float32)
    h = jnp.maximum(h + b1_ref[...], 0.0)
    out_ref[...] = (x + jnp.dot(h, w2_ref[...],
                                preferred_element_type=jnp.float32)
                    + b2_ref[...])


def _tc_mlp(x, sums_t, cnt_p, w1x, w1a, b1, w2, b2):
    return pl.pallas_call(
        _mlp_body,
        out_shape=jax.ShapeDtypeStruct((N_NODES, D_NODE), jnp.float32),
        grid=(-(-N_NODES // _NB),),
        in_specs=[
            pl.BlockSpec((_NB, D_NODE), lambda i: (i, 0)),
            pl.BlockSpec((NC, NS, _NB), lambda i: (0, 0, i)),
            pl.BlockSpec((NC, NS, _NB), lambda i: (0, 0, i)),
            pl.BlockSpec((D_NODE, HID), lambda i: (0, 0)),
            pl.BlockSpec((L, HID), lambda i: (0, 0)),
            pl.BlockSpec((1, HID), lambda i: (0, 0)),
            pl.BlockSpec((HID, D_NODE), lambda i: (0, 0)),
            pl.BlockSpec((1, D_NODE), lambda i: (0, 0)),
        ],
        out_specs=pl.BlockSpec((_NB, D_NODE), lambda i: (i, 0)),
    )(x, sums_t, cnt_p, w1x, w1a, b1, w2, b2)


@jax.jit
def kernel(x, edge_index, edge_attr, u, batch, W1, b1, W2, b2):
    ea_t = edge_attr.T
    sums_t, cnt_p = _sc_scatter_kernel()(ea_t, edge_index)
    return _tc_mlp(x, sums_t, cnt_p, W1[:D_NODE], W1[D_NODE:],
                   b1.reshape(1, HID), W2, b2.reshape(1, D_NODE))
